# baseline (speedup 1.0000x reference)
import jax
import jax.numpy as jnp
from jax.experimental import pallas as pl
from jax.experimental.pallas import tpu as pltpu

_INDEX = 50
_ROWS = 200
_FEAT = 64
_BAND = 8
_JBAND = _INDEX // _BAND
_LOCAL = _INDEX - _JBAND * _BAND
_BB = 512


def _scatter_band(x_ref, src_ref, o_ref):
    o_ref[...] = x_ref[...]
    o_ref[:, _LOCAL, :] = src_ref[...]


def kernel(x, src):
    b = x.shape[0]
    out = pl.pallas_call(
        _scatter_band,
        grid=(b // _BB,),
        in_specs=[
            pl.BlockSpec((_BB, _BAND, _FEAT), lambda i: (i, _JBAND, 0)),
            pl.BlockSpec((_BB, _FEAT), lambda i: (i, 0)),
        ],
        out_specs=pl.BlockSpec((_BB, _BAND, _FEAT), lambda i: (i, _JBAND, 0)),
        out_shape=jax.ShapeDtypeStruct((b, _ROWS, _FEAT), x.dtype),
        input_output_aliases={0: 0},
    )(x, src)
    return out
